# baseline (speedup 1.0000x reference)
import jax
import jax.numpy as jnp
from jax import lax
from jax.experimental import pallas as pl
from jax.experimental.pallas import tpu as pltpu
from jax.experimental.pallas import tpu_sc as plsc

B, C, T, D, V = 16, 8, 4096, 64, 2048
K = 128
NC, NS = 2, 16
NW = NC * NS
TOK_PER_W = (B * T) // NW
CHUNKS = TOK_PER_W // K


def _embed_body(x_hbm, emb_hbm, out_hbm, idx_v, rows0, rows1, outv0, outv1,
                sg0, sg1, so0, so1):
    cid = lax.axis_index("core")
    sid = lax.axis_index("sub")
    wid = sid * NC + cid
    b = wid // 2
    t_half = (wid % 2) * TOK_PER_W
    rows = (rows0, rows1)
    outv = (outv0, outv1)
    sg = (sg0, sg1)
    so = (so0, so1)

    tt0 = (wid % 2) * CHUNKS
    pltpu.sync_copy(x_hbm.at[b, pl.ds(tt0, CHUNKS)], idx_v)

    def off_body(tt, carry):
        for c in range(1, C):
            for j in range(K // 16):
                sl = pl.ds(j * 16, 16)
                idx_v[tt, c, sl] = idx_v[tt, c, sl] + (c * V)
        return carry
    lax.fori_loop(0, CHUNKS, off_body, 0)

    def fire_gathers(i, p):
        for c in range(C):
            pltpu.async_copy(
                emb_hbm.at[idx_v.at[i, c]], rows[p].at[c], sg[p])

    def drain_gathers(i, p):
        for c in range(C):
            pltpu.make_async_copy(
                emb_hbm.at[idx_v.at[i, c]], rows[p].at[c], sg[p]).wait()

    lane = lax.broadcasted_iota(jnp.int32, (16,), 0)
    dmid = lane % 8
    dhi_lo = [2 * h + lane // 8 for h in range(2)]
    dhi_hi = [4 + 2 * h + lane // 8 for h in range(2)]

    def accum(p):
        def tok_body(k, carry):
            acc_lo = [None] * 2
            acc_hi = [None] * 2
            for c in range(C):
                for h in range(2):
                    w = plsc.bitcast(rows[p][c, k, pl.ds(h * 32, 32)],
                                     jnp.int32)
                    a = plsc.bitcast(lax.shift_left(w, 16), jnp.float32)
                    bb = plsc.bitcast(
                        lax.bitwise_and(w, jnp.int32(-65536)), jnp.float32)
                    if c == 0:
                        acc_lo[h], acc_hi[h] = a, bb
                    else:
                        acc_lo[h] = acc_lo[h] + a
                        acc_hi[h] = acc_hi[h] + bb
            k_idx = jnp.full((16,), 0, jnp.int32) + k
            for h in range(2):
                plsc.store_scatter(outv[p], [dhi_lo[h], dmid, k_idx],
                                   acc_lo[h] * (1.0 / C))
                plsc.store_scatter(outv[p], [dhi_hi[h], dmid, k_idx],
                                   acc_hi[h] * (1.0 / C))
            return carry
        lax.fori_loop(0, K, tok_body, 0, unroll=2)

    def fire_out(i, p):
        pltpu.async_copy(outv[p].at[:, :, pl.ds(0, K)],
                         out_hbm.at[b, :, tt0 + i], so[p])

    def wait_out(i, p):
        pltpu.make_async_copy(
            outv[p].at[:, :, pl.ds(0, K)],
            out_hbm.at[b, :, tt0 + i], so[p]).wait()

    fire_gathers(0, 0)

    def pair_body(i, carry):
        ii = 2 * i
        fire_gathers(ii + 1, 1)
        drain_gathers(ii, 0)
        @pl.when(i > 0)
        def _():
            wait_out(ii - 2, 0)
        accum(0)
        fire_out(ii, 0)
        @pl.when(ii + 2 < CHUNKS)
        def _():
            fire_gathers(ii + 2, 0)
        drain_gathers(ii + 1, 1)
        @pl.when(i > 0)
        def _():
            wait_out(ii - 1, 1)
        accum(1)
        fire_out(ii + 1, 1)
        return carry

    lax.fori_loop(0, CHUNKS // 2, pair_body, 0)
    wait_out(CHUNKS - 2, 0)
    wait_out(CHUNKS - 1, 1)


_mesh = plsc.VectorSubcoreMesh(
    core_axis_name="core", subcore_axis_name="sub",
    num_cores=NC, num_subcores=NS)

_embed = pl.kernel(
    _embed_body,
    out_type=jax.ShapeDtypeStruct((B, D // 8, T // K, 8, K), jnp.float32),
    mesh=_mesh,
    scratch_types=[
        pltpu.VMEM((CHUNKS, C, K), jnp.int32),
        pltpu.VMEM((C, K, D), jnp.bfloat16),
        pltpu.VMEM((C, K, D), jnp.bfloat16),
        pltpu.VMEM((D // 8, 8, K + 1), jnp.float32),
        pltpu.VMEM((D // 8, 8, K + 1), jnp.float32),
        pltpu.SemaphoreType.DMA,
        pltpu.SemaphoreType.DMA,
        pltpu.SemaphoreType.DMA,
        pltpu.SemaphoreType.DMA,
    ],
    compiler_params=pltpu.CompilerParams(
        use_tc_tiling_on_sc=False, needs_layout_passes=False),
)


def kernel(x, emb):
    x4 = x.astype(jnp.int32).reshape(B, C, T // K, K).swapaxes(1, 2)
    emb_bf = (emb.astype(jnp.bfloat16)
              .reshape(C * V, 2, D // 2).swapaxes(1, 2).reshape(C * V, D))
    o5 = _embed(x4, emb_bf)
    return o5.transpose(0, 2, 4, 1, 3).reshape(B, T, D)


# speedup vs baseline: 1.0378x; 1.0378x over previous
import jax
import jax.numpy as jnp
from jax import lax
from jax.experimental import pallas as pl
from jax.experimental.pallas import tpu as pltpu
from jax.experimental.pallas import tpu_sc as plsc

B, C, T, D, V = 16, 8, 4096, 64, 2048
K = 128
NC, NS = 2, 16
NW = NC * NS
TOK_PER_W = (B * T) // NW
CHUNKS = TOK_PER_W // K


def _embed_body(x_hbm, emb_hbm, out_hbm, idx_v, rows0, rows1, outv0, outv1,
                sg0, sg1, so0, so1):
    cid = lax.axis_index("core")
    sid = lax.axis_index("sub")
    wid = sid * NC + cid
    b = wid // 2
    t_half = (wid % 2) * TOK_PER_W
    rows = (rows0, rows1)
    outv = (outv0, outv1)
    sg = (sg0, sg1)
    so = (so0, so1)

    tt0 = (wid % 2) * CHUNKS
    pltpu.sync_copy(x_hbm.at[b, pl.ds(tt0, CHUNKS)], idx_v)

    def off_body(tt, carry):
        for c in range(1, C):
            for j in range(K // 16):
                sl = pl.ds(j * 16, 16)
                idx_v[tt, c, sl] = idx_v[tt, c, sl] + (c * V)
        return carry
    lax.fori_loop(0, CHUNKS, off_body, 0)

    def fire_gathers(i, p):
        for c in range(C):
            pltpu.async_copy(
                emb_hbm.at[idx_v.at[i, c]], rows[p].at[c], sg[p])

    def drain_gathers(i, p):
        for c in range(C):
            pltpu.make_async_copy(
                emb_hbm.at[idx_v.at[i, c]], rows[p].at[c], sg[p]).wait()

    lane = lax.broadcasted_iota(jnp.int32, (16,), 0)
    dhi = [4 * g + lane // 4 for g in range(D // 32)]
    dmid_e = [(2 * lane) % 8 for g in range(D // 32)]
    dmid_o = [(2 * lane + 1) % 8 for g in range(D // 32)]

    def accum(p):
        def tok_body(k, carry):
            acc_e = [None] * (D // 32)
            acc_o = [None] * (D // 32)
            for c in range(C):
                for g in range(D // 32):
                    w = plsc.bitcast(rows[p][c, k, pl.ds(g * 32, 32)],
                                     jnp.int32)
                    a = plsc.bitcast(lax.shift_left(w, 16), jnp.float32)
                    bb = plsc.bitcast(
                        lax.bitwise_and(w, jnp.int32(-65536)), jnp.float32)
                    if c == 0:
                        acc_e[g], acc_o[g] = a, bb
                    else:
                        acc_e[g] = acc_e[g] + a
                        acc_o[g] = acc_o[g] + bb
            k_idx = jnp.full((16,), 0, jnp.int32) + k
            for g in range(D // 32):
                plsc.store_scatter(outv[p], [dhi[g], dmid_e[g], k_idx],
                                   acc_e[g] * (1.0 / C))
                plsc.store_scatter(outv[p], [dhi[g], dmid_o[g], k_idx],
                                   acc_o[g] * (1.0 / C))
            return carry
        lax.fori_loop(0, K, tok_body, 0, unroll=2)

    def fire_out(i, p):
        pltpu.async_copy(outv[p].at[:, :, pl.ds(0, K)],
                         out_hbm.at[b, :, tt0 + i], so[p])

    def wait_out(i, p):
        pltpu.make_async_copy(
            outv[p].at[:, :, pl.ds(0, K)],
            out_hbm.at[b, :, tt0 + i], so[p]).wait()

    fire_gathers(0, 0)

    def pair_body(i, carry):
        ii = 2 * i
        fire_gathers(ii + 1, 1)
        drain_gathers(ii, 0)
        @pl.when(i > 0)
        def _():
            wait_out(ii - 2, 0)
        accum(0)
        fire_out(ii, 0)
        @pl.when(ii + 2 < CHUNKS)
        def _():
            fire_gathers(ii + 2, 0)
        drain_gathers(ii + 1, 1)
        @pl.when(i > 0)
        def _():
            wait_out(ii - 1, 1)
        accum(1)
        fire_out(ii + 1, 1)
        return carry

    lax.fori_loop(0, CHUNKS // 2, pair_body, 0)
    wait_out(CHUNKS - 2, 0)
    wait_out(CHUNKS - 1, 1)


_mesh = plsc.VectorSubcoreMesh(
    core_axis_name="core", subcore_axis_name="sub",
    num_cores=NC, num_subcores=NS)

_embed = pl.kernel(
    _embed_body,
    out_type=jax.ShapeDtypeStruct((B, D // 8, T // K, 8, K), jnp.float32),
    mesh=_mesh,
    scratch_types=[
        pltpu.VMEM((CHUNKS, C, K), jnp.int32),
        pltpu.VMEM((C, K, D), jnp.bfloat16),
        pltpu.VMEM((C, K, D), jnp.bfloat16),
        pltpu.VMEM((D // 8, 8, K + 1), jnp.float32),
        pltpu.VMEM((D // 8, 8, K + 1), jnp.float32),
        pltpu.SemaphoreType.DMA,
        pltpu.SemaphoreType.DMA,
        pltpu.SemaphoreType.DMA,
        pltpu.SemaphoreType.DMA,
    ],
    compiler_params=pltpu.CompilerParams(
        use_tc_tiling_on_sc=False, needs_layout_passes=False),
)


def kernel(x, emb):
    x4 = x.astype(jnp.int32).reshape(B, C, T // K, K).swapaxes(1, 2)
    o5 = _embed(x4, emb.astype(jnp.bfloat16))
    return o5.transpose(0, 2, 4, 1, 3).reshape(B, T, D)
